# baseline (speedup 1.0000x reference)
import jax
import jax.numpy as jnp
from jax.experimental import pallas as pl
from jax.experimental.pallas import tpu as pltpu

_PRE_NMS_THRESH = 0.05
_PRE_NMS_TOPK = 1000
_POST_NMS_TOPK = 100
_NMS_THRESH = 0.6
_K = 1024
_RB = 128


def _nms_kernel(bt_ref, bn_ref, st_ref, out_ref, adj_scr):
    x1 = bt_ref[0:1, :]
    y1 = bt_ref[1:2, :]
    x2 = bt_ref[2:3, :]
    y2 = bt_ref[3:4, :]
    area = (x2 - x1) * (y2 - y1)

    def adj_block(b, carry):
        r0 = b * _RB
        x1c = bn_ref[pl.ds(r0, _RB), 0:1]
        y1c = bn_ref[pl.ds(r0, _RB), 1:2]
        x2c = bn_ref[pl.ds(r0, _RB), 2:3]
        y2c = bn_ref[pl.ds(r0, _RB), 3:4]
        areac = (x2c - x1c) * (y2c - y1c)
        w = jnp.clip(jnp.minimum(x2c, x2) - jnp.maximum(x1c, x1), 0.0)
        h = jnp.clip(jnp.minimum(y2c, y2) - jnp.maximum(y1c, y1), 0.0)
        inter = w * h
        iou = inter / jnp.maximum(areac + area - inter, 1e-9)
        sub = jax.lax.broadcasted_iota(jnp.int32, (_RB, _K), 0) + r0
        lane = jax.lax.broadcasted_iota(jnp.int32, (_RB, _K), 1)
        adj = jnp.where((iou > _NMS_THRESH) & (sub < lane), 1.0, 0.0)
        adj_scr[pl.ds(r0, _RB), :] = adj
        return carry

    jax.lax.fori_loop(0, _K // _RB, adj_block, 0)

    def cond(state):
        _, changed = state
        return changed

    def body(state):
        x, _ = state
        cnt = jax.lax.dot_general(x, adj_scr[...],
                                  (((1,), (0,)), ((), ())),
                                  preferred_element_type=jnp.float32)
        xn = jnp.where(cnt > 0.5, 0.0, 1.0)
        return xn, jnp.any(xn != x)

    x0 = jnp.ones((1, _K), jnp.float32)
    keep, _ = jax.lax.while_loop(cond, body, (x0, True))
    out_ref[...] = keep * st_ref[...]


def kernel(boxes, scores):
    probs = jax.nn.sigmoid(scores)
    masked = jnp.where(probs > _PRE_NMS_THRESH, probs, 0.0)
    top_scores, top_idx = jax.lax.top_k(masked, _PRE_NMS_TOPK)
    top_boxes = jnp.take(boxes, top_idx, axis=0)
    pad = _K - _PRE_NMS_TOPK
    bn = jnp.pad(top_boxes, ((0, pad), (0, 4)))
    bt = jnp.pad(top_boxes.T, ((0, 4), (0, pad)))
    st = jnp.pad(top_scores, (0, pad))[None, :]
    kept = pl.pallas_call(
        _nms_kernel,
        out_shape=jax.ShapeDtypeStruct((1, _K), jnp.float32),
        scratch_shapes=[pltpu.VMEM((_K, _K), jnp.float32)],
    )(bt, bn, st)
    kept_scores = kept[0, :_PRE_NMS_TOPK]
    final_scores, final_idx = jax.lax.top_k(kept_scores, _POST_NMS_TOPK)
    final_boxes = jnp.take(top_boxes, final_idx, axis=0)
    return jnp.concatenate([final_boxes, final_scores[:, None]], axis=1)
